# Initial kernel scaffold; baseline (speedup 1.0000x reference)
import jax
import jax.numpy as jnp
from jax.experimental import pallas as pl


def kernel(tokens, token_embedding, position_embedding):
    raise NotImplementedError("write your pallas kernel here")


# speedup vs baseline: 2.3856x; 2.3856x over previous
import functools

import jax
import jax.numpy as jnp
from jax import lax
from jax.experimental import pallas as pl
from jax.experimental.pallas import tpu as pltpu
from jax.experimental.pallas import tpu_sc as plsc

N_VOCAB = 100000
N_EMBD = 64
N_TOKEN = 200
BATCH = 4096

NC = 2
NS = 16
NW = NC * NS
B_FLAT = BATCH * N_TOKEN
B_PER_W = B_FLAT // NW
SEQ_PER_W = B_PER_W // N_TOKEN
LANES = 16
VPR = N_EMBD // LANES


def _emb_kernel(table_hbm, idx_hbm, pos_hbm, out_hbm, idx_v, pos_v, buf, sem, osem):
    wid = lax.axis_index("s") * NC + lax.axis_index("c")
    base = wid * B_PER_W

    pltpu.sync_copy(idx_hbm.at[pl.ds(base, B_PER_W)], idx_v)
    pltpu.sync_copy(pos_hbm, pos_v)

    def seq_body(s, carry):
        o = s * N_TOKEN
        cp1 = pltpu.make_async_copy(
            table_hbm.at[idx_v.at[pl.ds(o, 128)]], buf.at[pl.ds(0, 128)], sem)
        cp2 = pltpu.make_async_copy(
            table_hbm.at[idx_v.at[pl.ds(o + 128, 72)]], buf.at[pl.ds(128, 72)], sem)
        cp1.start()
        cp2.start()
        cp1.wait()
        cp2.wait()

        def add_body(r, c2):
            for c in range(VPR):
                sl = pl.ds(c * LANES, LANES)
                buf[r, sl] = buf[r, sl] + pos_v[r, sl]
            return c2
        lax.fori_loop(0, N_TOKEN, add_body, 0, unroll=2)

        pltpu.sync_copy(buf, out_hbm.at[pl.ds(base + o, N_TOKEN)])
        return carry

    lax.fori_loop(0, SEQ_PER_W, seq_body, 0)


@jax.jit
def _emb(table, idx_flat, pos):
    mesh = plsc.VectorSubcoreMesh(core_axis_name="c", subcore_axis_name="s")
    f = pl.kernel(
        _emb_kernel,
        out_type=jax.ShapeDtypeStruct((B_FLAT, N_EMBD), jnp.float32),
        mesh=mesh,
        scratch_types=[
            pltpu.VMEM((B_PER_W,), jnp.int32),
            pltpu.VMEM((N_TOKEN, N_EMBD), jnp.float32),
            pltpu.VMEM((N_TOKEN, N_EMBD), jnp.float32),
            pltpu.SemaphoreType.DMA,
            pltpu.SemaphoreType.DMA,
        ],
        compiler_params=pltpu.CompilerParams(use_tc_tiling_on_sc=False),
    )
    return f(table, idx_flat, pos)


def kernel(tokens, token_embedding, position_embedding):
    idx_flat = tokens.reshape(B_FLAT)
    out = _emb(token_embedding, idx_flat, position_embedding)
    return out.reshape(BATCH, N_TOKEN, N_EMBD)
